# baseline (device time: 12407 ns/iter reference)
import jax
import jax.numpy as jnp
from jax import lax
from jax.experimental import pallas as pl
from jax.experimental.pallas import tpu as pltpu

M = 512
N = 1024
HALF = N // 2

XB = 128
YB = 48
REST = M - 2 * XB - 2 * YB
DIRECT = XB + YB + REST
XC = (0, 32, 64, 96, 128)
NX = len(XC) - 1
XOFF = YB
ROFF = YB + XB


def kernel(x):
    def body(
        x_ref,
        out_ref,
        send_z,
        recv_z,
        recv_x,
        recv_y,
        z_send_sems,
        z_recv_sems,
        x_send_sems,
        x_recv_sems,
        y_send_sem,
        y_recv_sem,
    ):
        my_x = lax.axis_index("x")
        my_y = lax.axis_index("y")
        my_z = lax.axis_index("z")
        peer_z = (my_x, my_y, 1 - my_z)
        peer_x = (1 - my_x, my_y, my_z)
        peer_y = (my_x, 1 - my_y, my_z)

        parity = jnp.bitwise_xor(my_x, my_y)
        own_xb = (1 - my_x) * XB
        miss_xb = my_x * XB
        own_yb = 2 * XB + YB * (1 - parity)
        miss_yb = 2 * XB + YB * parity
        rest_row = 2 * XB + 2 * YB

        barrier_sem = pltpu.get_barrier_semaphore()
        for nbr in (peer_z, peer_x, peer_y):
            pl.semaphore_signal(
                barrier_sem,
                inc=1,
                device_id=nbr,
                device_id_type=pl.DeviceIdType.MESH,
            )

        other_off = (1 - my_z) * HALF
        send_z[0:XOFF] = x_ref[0, pl.ds(own_yb, YB), pl.ds(other_off, HALF)].astype(
            jnp.bfloat16
        )
        send_z[XOFF:ROFF] = x_ref[
            0, pl.ds(own_xb, XB), pl.ds(other_off, HALF)
        ].astype(jnp.bfloat16)
        pl.semaphore_wait(barrier_sem, 3)

        def z_chunk(lo, hi, sem_idx):
            return pltpu.make_async_remote_copy(
                src_ref=send_z.at[lo:hi],
                dst_ref=recv_z.at[lo:hi],
                send_sem=z_send_sems.at[sem_idx],
                recv_sem=z_recv_sems.at[sem_idx],
                device_id=peer_z,
                device_id_type=pl.DeviceIdType.MESH,
            )

        zy = z_chunk(0, XOFF, 0)
        zy.start()
        zx = []
        for i in range(NX):
            r = z_chunk(XOFF + XC[i], XOFF + XC[i + 1], 1 + i)
            r.start()
            zx.append(r)

        send_z[ROFF:DIRECT] = x_ref[
            0, pl.ds(rest_row, REST), pl.ds(other_off, HALF)
        ].astype(jnp.bfloat16)
        zrest = z_chunk(ROFF, DIRECT, 1 + NX)
        zrest.start()

        my_off = my_z * HALF
        out_ref[...] = x_ref[0, :, pl.ds(my_off, HALF)]

        zy.wait()
        fy = pltpu.make_async_remote_copy(
            src_ref=recv_z.at[0:XOFF],
            dst_ref=recv_y.at[0:YB],
            send_sem=y_send_sem,
            recv_sem=y_recv_sem,
            device_id=peer_y,
            device_id_type=pl.DeviceIdType.MESH,
        )
        fy.start()

        fxs = []
        for i in range(NX):
            zx[i].wait()
            f = pltpu.make_async_remote_copy(
                src_ref=recv_z.at[XOFF + XC[i] : XOFF + XC[i + 1]],
                dst_ref=recv_x.at[XC[i] : XC[i + 1]],
                send_sem=x_send_sems.at[i],
                recv_sem=x_recv_sems.at[i],
                device_id=peer_x,
                device_id_type=pl.DeviceIdType.MESH,
            )
            f.start()
            fxs.append(f)

        zrest.wait()
        out_ref[pl.ds(own_yb, YB)] = out_ref[pl.ds(own_yb, YB)] + recv_z[
            0:XOFF
        ].astype(jnp.float32)
        out_ref[pl.ds(own_xb, XB)] = out_ref[pl.ds(own_xb, XB)] + recv_z[
            XOFF:ROFF
        ].astype(jnp.float32)
        out_ref[pl.ds(rest_row, REST)] = out_ref[pl.ds(rest_row, REST)] + recv_z[
            ROFF:DIRECT
        ].astype(jnp.float32)

        fy.wait_send()
        for i in range(NX):
            fxs[i].wait_send()
        for i in range(NX):
            rx = pltpu.make_async_remote_copy(
                src_ref=recv_x.at[XC[i] : XC[i + 1]],
                dst_ref=recv_x.at[XC[i] : XC[i + 1]],
                send_sem=x_send_sems.at[i],
                recv_sem=x_recv_sems.at[i],
                device_id=peer_x,
                device_id_type=pl.DeviceIdType.MESH,
            )
            rx.wait_recv()
        out_ref[pl.ds(miss_xb, XB)] = out_ref[pl.ds(miss_xb, XB)] + recv_x[
            ...
        ].astype(jnp.float32)
        ry = pltpu.make_async_remote_copy(
            src_ref=recv_y.at[0:YB],
            dst_ref=recv_y.at[0:YB],
            send_sem=y_send_sem,
            recv_sem=y_recv_sem,
            device_id=peer_y,
            device_id_type=pl.DeviceIdType.MESH,
        )
        ry.wait_recv()
        out_ref[pl.ds(miss_yb, YB)] = out_ref[pl.ds(miss_yb, YB)] + recv_y[
            ...
        ].astype(jnp.float32)

    return pl.pallas_call(
        body,
        out_shape=jax.ShapeDtypeStruct((M, HALF), jnp.float32),
        in_specs=[pl.BlockSpec(memory_space=pltpu.VMEM)],
        out_specs=pl.BlockSpec(memory_space=pltpu.VMEM),
        scratch_shapes=[
            pltpu.VMEM((DIRECT, HALF), jnp.bfloat16),
            pltpu.VMEM((DIRECT, HALF), jnp.bfloat16),
            pltpu.VMEM((XB, HALF), jnp.bfloat16),
            pltpu.VMEM((YB, HALF), jnp.bfloat16),
            pltpu.SemaphoreType.DMA((2 + NX,)),
            pltpu.SemaphoreType.DMA((2 + NX,)),
            pltpu.SemaphoreType.DMA((NX,)),
            pltpu.SemaphoreType.DMA((NX,)),
            pltpu.SemaphoreType.DMA,
            pltpu.SemaphoreType.DMA,
        ],
        compiler_params=pltpu.CompilerParams(collective_id=0),
    )(x)


# device time: 11875 ns/iter; 1.0448x vs baseline; 1.0448x over previous
import jax
import jax.numpy as jnp
from jax import lax
from jax.experimental import pallas as pl
from jax.experimental.pallas import tpu as pltpu

M = 512
N = 1024
HALF = N // 2

F = 160
FWD_CUTS = (0, 48, 96, 128, 160)
REST = M - 2 * F
DIRECT = F + REST
NF = len(FWD_CUTS) - 1


def kernel(x):
    def body(
        x_ref,
        out_ref,
        send_z,
        recv_z,
        recv_x,
        z_send_sems,
        z_recv_sems,
        x_send_sems,
        x_recv_sems,
    ):
        my_x = lax.axis_index("x")
        my_y = lax.axis_index("y")
        my_z = lax.axis_index("z")
        peer_z = (my_x, my_y, 1 - my_z)
        peer_x = (1 - my_x, my_y, my_z)

        barrier_sem = pltpu.get_barrier_semaphore()
        for nbr in (peer_z, peer_x):
            pl.semaphore_signal(
                barrier_sem,
                inc=1,
                device_id=nbr,
                device_id_type=pl.DeviceIdType.MESH,
            )

        other_off = (1 - my_z) * HALF
        fwd_row = my_x * F
        send_z[0:F] = x_ref[0, pl.ds(fwd_row, F), pl.ds(other_off, HALF)].astype(
            jnp.bfloat16
        )
        pl.semaphore_wait(barrier_sem, 2)

        def z_chunk(lo, hi, sem_idx):
            return pltpu.make_async_remote_copy(
                src_ref=send_z.at[lo:hi],
                dst_ref=recv_z.at[lo:hi],
                send_sem=z_send_sems.at[sem_idx],
                recv_sem=z_recv_sems.at[sem_idx],
                device_id=peer_z,
                device_id_type=pl.DeviceIdType.MESH,
            )

        z_fwd = []
        for i in range(NF):
            r = z_chunk(FWD_CUTS[i], FWD_CUTS[i + 1], i)
            r.start()
            z_fwd.append(r)

        send_z[F:DIRECT] = x_ref[
            0, pl.ds(2 * F, REST), pl.ds(other_off, HALF)
        ].astype(jnp.bfloat16)
        zc = z_chunk(F, DIRECT, NF)
        zc.start()

        my_off = my_z * HALF
        out_ref[...] = x_ref[0, :, pl.ds(my_off, HALF)]

        def x_fwd(lo, hi, sem_idx):
            return pltpu.make_async_remote_copy(
                src_ref=recv_z.at[lo:hi],
                dst_ref=recv_x.at[lo:hi],
                send_sem=x_send_sems.at[sem_idx],
                recv_sem=x_recv_sems.at[sem_idx],
                device_id=peer_x,
                device_id_type=pl.DeviceIdType.MESH,
            )

        fwds = []
        for i in range(NF):
            z_fwd[i].wait()
            f = x_fwd(FWD_CUTS[i], FWD_CUTS[i + 1], i)
            f.start()
            fwds.append(f)

        zc.wait()
        out_ref[pl.ds(fwd_row, F)] = out_ref[pl.ds(fwd_row, F)] + recv_z[
            0:F
        ].astype(jnp.float32)
        out_ref[2 * F : M] = out_ref[2 * F : M] + recv_z[F:DIRECT].astype(
            jnp.float32
        )

        for i in range(NF):
            fwds[i].wait_send()
        for i in range(NF):
            rx = pltpu.make_async_remote_copy(
                src_ref=recv_x.at[FWD_CUTS[i] : FWD_CUTS[i + 1]],
                dst_ref=recv_x.at[FWD_CUTS[i] : FWD_CUTS[i + 1]],
                send_sem=x_send_sems.at[i],
                recv_sem=x_recv_sems.at[i],
                device_id=peer_x,
                device_id_type=pl.DeviceIdType.MESH,
            )
            rx.wait_recv()
        nbr_row = (1 - my_x) * F
        out_ref[pl.ds(nbr_row, F)] = out_ref[pl.ds(nbr_row, F)] + recv_x[
            0:F
        ].astype(jnp.float32)

    return pl.pallas_call(
        body,
        out_shape=jax.ShapeDtypeStruct((M, HALF), jnp.float32),
        in_specs=[pl.BlockSpec(memory_space=pltpu.VMEM)],
        out_specs=pl.BlockSpec(memory_space=pltpu.VMEM),
        scratch_shapes=[
            pltpu.VMEM((DIRECT, HALF), jnp.bfloat16),
            pltpu.VMEM((DIRECT, HALF), jnp.bfloat16),
            pltpu.VMEM((F, HALF), jnp.bfloat16),
            pltpu.SemaphoreType.DMA((NF + 1,)),
            pltpu.SemaphoreType.DMA((NF + 1,)),
            pltpu.SemaphoreType.DMA((NF,)),
            pltpu.SemaphoreType.DMA((NF,)),
        ],
        compiler_params=pltpu.CompilerParams(collective_id=0),
    )(x)
